# baseline (speedup 1.0000x reference)
import jax
import jax.numpy as jnp
from jax.experimental import pallas as pl
from jax.experimental.pallas import tpu as pltpu

B, L, O, DM = 8, 48, 100, 128
OP = 112
LT = 8
PLT = 16

_f32 = jnp.float32
_bf16 = jnp.bfloat16


def _prep_step(x_ref, x16_ref, tdn_ref):
    zero_tile = jnp.zeros((OP, OP), _bf16)
    zero_row = jnp.zeros((1, OP), _f32)
    for t in range(PLT):
        x = x_ref[t]
        x16_ref[t] = zero_tile
        x16_ref[t, :O, :O] = x.astype(_bf16)
        td_row = jnp.sum(x, axis=1, keepdims=True).T
        td = jnp.sum(x, axis=0, keepdims=True) + td_row
        tdn_ref[t] = zero_row
        tdn_ref[t, :, :O] = td / jnp.sum(td)


def _main_step(x_ref, tdn_ref, dis_ref, w1_ref, b1_ref, w2_ref, b2_ref,
               out_ref, disw_ref, w1s_ref):
    @pl.when(pl.program_id(0) == 0)
    def _init():
        dis = dis_ref[...]
        sd = jnp.sqrt(dis)
        dw = jnp.where(dis <= 2.0, sd, 0.0) / jnp.sum(sd, axis=1,
                                                      keepdims=True)
        disw_ref[...] = jnp.zeros((OP, OP), _bf16)
        disw_ref[:O, :O] = dw.astype(_bf16)
        w1 = w1_ref[...]
        w1s_ref[...] = jnp.zeros((2 * OP, DM), _bf16)
        w1s_ref[:O] = w1[:O].astype(_bf16)
        w1s_ref[OP:OP + O] = w1[O:].astype(_bf16)

    w1s = w1s_ref[...]
    w2 = w2_ref[...].astype(_bf16)
    b1v = b1_ref[0]
    b2v = b2_ref[0]
    disw = disw_ref[...]

    x3 = x_ref[...]
    xts = [x3[t].T for t in range(LT)]
    xtall = jnp.concatenate(xts, axis=0)
    xall = x3.reshape(LT * OP, OP)

    yall = (jnp.dot(xall, w1s[:OP], preferred_element_type=_f32)
            + jnp.dot(xtall, w1s[OP:], preferred_element_type=_f32)
            + b1v)
    y16 = yall.astype(_bf16)

    ycat = jnp.concatenate([y16[t * OP:(t + 1) * OP] for t in range(LT)],
                           axis=1)
    fall = jnp.dot(disw, ycat, preferred_element_type=_f32)

    geo_in = []
    sem_in = []
    for t in range(LT):
        sl = slice(t * OP, (t + 1) * OP)
        tdn16 = tdn_ref[t].astype(_bf16)
        deg_w = jnp.where((x3[t] > 0) | (xts[t] > 0),
                          jnp.broadcast_to(tdn16, (OP, OP)),
                          _bf16(0))
        bt = jnp.dot(deg_w, y16[sl], preferred_element_type=_f32)
        yt = yall[sl]
        geo_in.append((yt + fall[:, t * DM:(t + 1) * DM]).astype(_bf16))
        sem_in.append((yt + bt).astype(_bf16))

    geo_all = jnp.dot(jnp.concatenate(geo_in, axis=0), w2,
                      preferred_element_type=_f32) + b2v
    sem_all = jnp.dot(jnp.concatenate(sem_in, axis=0), w2,
                      preferred_element_type=_f32) + b2v
    for t in range(LT):
        out_ref[t] = jnp.concatenate([geo_all[t * OP:t * OP + O],
                                      sem_all[t * OP:t * OP + O]], axis=-1)


def kernel(X, dis_matrix, W1, b1, W2, b2):
    Bx, Lx, Ox, _ = X.shape
    n = Bx * Lx
    Xr = X.reshape(n, Ox, Ox)

    x16, tdn = pl.pallas_call(
        _prep_step,
        grid=(n // PLT,),
        in_specs=[pl.BlockSpec((PLT, Ox, Ox), lambda i: (i, 0, 0))],
        out_specs=[pl.BlockSpec((PLT, OP, OP), lambda i: (i, 0, 0)),
                   pl.BlockSpec((PLT, 1, OP), lambda i: (i, 0, 0))],
        out_shape=[jax.ShapeDtypeStruct((n, OP, OP), _bf16),
                   jax.ShapeDtypeStruct((n, 1, OP), _f32)],
    )(Xr)

    out = pl.pallas_call(
        _main_step,
        grid=(n // LT,),
        in_specs=[
            pl.BlockSpec((LT, OP, OP), lambda i: (i, 0, 0)),
            pl.BlockSpec((LT, 1, OP), lambda i: (i, 0, 0)),
            pl.BlockSpec((Ox, Ox), lambda i: (0, 0)),
            pl.BlockSpec((2 * Ox, DM), lambda i: (0, 0)),
            pl.BlockSpec((1, DM), lambda i: (0, 0)),
            pl.BlockSpec((DM, DM), lambda i: (0, 0)),
            pl.BlockSpec((1, DM), lambda i: (0, 0)),
        ],
        out_specs=pl.BlockSpec((LT, Ox, 2 * DM), lambda i: (i, 0, 0)),
        out_shape=jax.ShapeDtypeStruct((n, Ox, 2 * DM), _f32),
        scratch_shapes=[pltpu.VMEM((OP, OP), _bf16),
                        pltpu.VMEM((2 * OP, DM), _bf16)],
    )(x16, tdn, dis_matrix, W1, b1.reshape(1, DM), W2, b2.reshape(1, DM))
    return out.reshape(Bx, Lx, Ox, 2 * DM)


# speedup vs baseline: 1.3484x; 1.3484x over previous
import jax
import jax.numpy as jnp
from jax.experimental import pallas as pl
from jax.experimental.pallas import tpu as pltpu

B, L, O, DM = 8, 48, 100, 128
OP = 112
LT = 8

_f32 = jnp.float32
_bf16 = jnp.bfloat16


def _main_step(x_ref, dis_ref, w1_ref, b1_ref, w2_ref, b2_ref,
               out_ref, disw_ref, w1s_ref, x16_ref, xt16_ref, tdn_ref):
    @pl.when(pl.program_id(0) == 0)
    def _init():
        dis = dis_ref[...]
        sd = jnp.sqrt(dis)
        dw = jnp.where(dis <= 2.0, sd, 0.0) / jnp.sum(sd, axis=1,
                                                      keepdims=True)
        disw_ref[...] = jnp.zeros((OP, OP), _bf16)
        disw_ref[:O, :O] = dw.astype(_bf16)
        w1 = w1_ref[...]
        w1s_ref[...] = jnp.zeros((2 * OP, DM), _bf16)
        w1s_ref[:O] = w1[:O].astype(_bf16)
        w1s_ref[OP:OP + O] = w1[O:].astype(_bf16)
        x16_ref[...] = jnp.zeros((LT, OP, OP), _bf16)
        xt16_ref[...] = jnp.zeros((LT, OP, OP), _bf16)

    w1s = w1s_ref[...]
    w2 = w2_ref[...].astype(_bf16)
    b1v = b1_ref[0]
    b2v = b2_ref[0]
    disw = disw_ref[...]

    x3 = x_ref[...]
    xts = [x3[t].T for t in range(LT)]
    x16_ref[:, :O, :O] = x3.astype(_bf16)
    for t in range(LT):
        xt16_ref[t, :O, :O] = xts[t].astype(_bf16)

    td = jnp.sum(x3, axis=1) + jnp.sum(jnp.stack(xts), axis=1)
    tdn_ref[:, :O] = td / jnp.sum(td, axis=1, keepdims=True)

    x16_3 = x16_ref[...]
    xt16_3 = xt16_ref[...]
    xall = x16_3.reshape(LT * OP, OP)
    xtall = xt16_3.reshape(LT * OP, OP)
    yall = (jnp.dot(xall, w1s[:OP], preferred_element_type=_f32)
            + jnp.dot(xtall, w1s[OP:], preferred_element_type=_f32)
            + b1v)
    y16 = yall.astype(_bf16)

    ycat = jnp.concatenate([y16[t * OP:(t + 1) * OP] for t in range(LT)],
                           axis=1)
    fall = jnp.dot(disw, ycat, preferred_element_type=_f32)

    tdnv = tdn_ref[...]
    geo_in = []
    sem_in = []
    for t in range(LT):
        sl = slice(t * OP, (t + 1) * OP)
        tdn16 = tdnv[t:t + 1].astype(_bf16)
        deg_w = jnp.where((x16_3[t] > 0) | (xt16_3[t] > 0),
                          jnp.broadcast_to(tdn16, (OP, OP)),
                          _bf16(0))
        bt = jnp.dot(deg_w, y16[sl], preferred_element_type=_f32)
        yt = yall[sl]
        geo_in.append((yt + fall[:, t * DM:(t + 1) * DM]).astype(_bf16))
        sem_in.append((yt + bt).astype(_bf16))

    geo_all = jnp.dot(jnp.concatenate(geo_in, axis=0), w2,
                      preferred_element_type=_f32) + b2v
    sem_all = jnp.dot(jnp.concatenate(sem_in, axis=0), w2,
                      preferred_element_type=_f32) + b2v
    for t in range(LT):
        out_ref[t] = jnp.concatenate([geo_all[t * OP:t * OP + O],
                                      sem_all[t * OP:t * OP + O]], axis=-1)


def kernel(X, dis_matrix, W1, b1, W2, b2):
    Bx, Lx, Ox, _ = X.shape
    n = Bx * Lx
    Xr = X.reshape(n, Ox, Ox)

    out = pl.pallas_call(
        _main_step,
        grid=(n // LT,),
        in_specs=[
            pl.BlockSpec((LT, Ox, Ox), lambda i: (i, 0, 0)),
            pl.BlockSpec((Ox, Ox), lambda i: (0, 0)),
            pl.BlockSpec((2 * Ox, DM), lambda i: (0, 0)),
            pl.BlockSpec((1, DM), lambda i: (0, 0)),
            pl.BlockSpec((DM, DM), lambda i: (0, 0)),
            pl.BlockSpec((1, DM), lambda i: (0, 0)),
        ],
        out_specs=pl.BlockSpec((LT, Ox, 2 * DM), lambda i: (i, 0, 0)),
        out_shape=jax.ShapeDtypeStruct((n, Ox, 2 * DM), _f32),
        scratch_shapes=[pltpu.VMEM((OP, OP), _bf16),
                        pltpu.VMEM((2 * OP, DM), _bf16),
                        pltpu.VMEM((LT, OP, OP), _bf16),
                        pltpu.VMEM((LT, OP, OP), _bf16),
                        pltpu.VMEM((LT, OP), _f32)],
    )(Xr, dis_matrix, W1, b1.reshape(1, DM), W2, b2.reshape(1, DM))
    return out.reshape(Bx, Lx, Ox, 2 * DM)
